# baseline (speedup 1.0000x reference)
import jax
import jax.numpy as jnp
from jax import lax
from jax.experimental import pallas as pl
from jax.experimental.pallas import tpu as pltpu
from jax.experimental.pallas import tpu_sc as plsc

B = 16384
L = 200
VOCAB = 1000000
D = 64
NUM_CLASSES = 2

NC = 2
NS = 16
NW = NC * NS

TVC = 256
H = 500224
NT = H // TVC


def _tr_body(a_ref, b_ref, out_ref):
    out_ref[:, 0:D] = a_ref[...].T
    out_ref[:, D:2 * D] = b_ref[...].T


@jax.jit
def _transpose(tabt):
    return pl.pallas_call(
        _tr_body,
        grid=(NT,),
        in_specs=[
            pl.BlockSpec((D, TVC), lambda j: (0, j)),
            pl.BlockSpec(
                (D, TVC), lambda j: (0, jnp.minimum(j + NT, VOCAB // TVC))
            ),
        ],
        out_specs=pl.BlockSpec((TVC, 2 * D), lambda j: (j, 0)),
        out_shape=jax.ShapeDtypeStruct((H, 2 * D), jnp.float32),
    )(tabt, tabt)


RPW = B // NW
CB = 4
G = 100
GPB = L // G
NGC = CB * GPB
NCHUNK = RPW // CB


def _gather_mean_body(x_hbm, tab_hbm, out_hbm, idx_v, rows_v, acc_v, sem0, sem1):
    cid = lax.axis_index("c")
    sid = lax.axis_index("s")
    wid = sid * NC + cid
    base = wid * RPW
    sems = (sem0, sem1)

    def issue(c, p):
        b0 = base + c * CB
        pltpu.sync_copy(x_hbm.at[pl.ds(GPB * b0, NGC)], idx_v.at[p])
        for g in range(NGC):
            pltpu.async_copy(
                tab_hbm.at[idx_v.at[p, g]],
                rows_v.at[p, pl.ds(g * G, G)],
                sems[p],
            )

    def wait_all(p):
        for g in range(NGC):
            pltpu.make_async_copy(
                tab_hbm.at[idx_v.at[p, g]],
                rows_v.at[p, pl.ds(g * G, G)],
                sems[p],
            ).wait()

    def compute(c, p):
        b0 = base + c * CB
        for b in range(CB):
            accs = [jnp.zeros((16,), jnp.float32) for _ in range(4)]

            def red(jj, a, b=b, p=p):
                r = b * L + jj * 4
                out = []
                for q in range(4):
                    r0 = rows_v[p, r, pl.ds(q * 16, 16)]
                    r1 = rows_v[p, r + 1, pl.ds(q * 16, 16)]
                    r2 = rows_v[p, r + 2, pl.ds(q * 16, 16)]
                    r3 = rows_v[p, r + 3, pl.ds(q * 16, 16)]
                    out.append(a[q] + ((r0 + r1) + (r2 + r3)))
                return out

            accs = lax.fori_loop(0, L // 4, red, accs)
            for q in range(4):
                acc_v[b, pl.ds(q * 16, 16)] = accs[q] / jnp.float32(L)
        pltpu.sync_copy(acc_v, out_hbm.at[pl.ds(b0, CB)])

    issue(0, 0)

    def body(i, carry):
        c0 = 2 * i
        issue(c0 + 1, 1)
        wait_all(0)
        compute(c0, 0)

        @pl.when(c0 + 2 < NCHUNK)
        def _():
            issue(c0 + 2, 0)

        wait_all(1)
        compute(c0 + 1, 1)
        return carry

    lax.fori_loop(0, NCHUNK // 2, body, 0)


@jax.jit
def _gather_mean(x2d, table):
    mesh = plsc.VectorSubcoreMesh(core_axis_name="c", subcore_axis_name="s")
    f = pl.kernel(
        _gather_mean_body,
        out_type=jax.ShapeDtypeStruct((B, D), jnp.float32),
        mesh=mesh,
        scratch_types=[
            pltpu.VMEM((2, NGC, G), jnp.int32),
            pltpu.VMEM((2, CB * L, D), jnp.float32),
            pltpu.VMEM((CB, D), jnp.float32),
            pltpu.SemaphoreType.DMA,
            pltpu.SemaphoreType.DMA,
        ],
        compiler_params=pltpu.CompilerParams(use_tc_tiling_on_sc=False),
    )
    return f(x2d, table)


BT = 2048


def _mlp_body(h_ref, w1t_ref, b1_ref, w2t_ref, b2_ref, probs_ref, cls_ref):
    h = h_ref[...]
    z = jnp.tanh(jnp.dot(h, w1t_ref[...]) + b1_ref[...])
    logits = jnp.tanh(jnp.dot(z, w2t_ref[...]) + b2_ref[...])
    m = jnp.max(logits, axis=1, keepdims=True)
    e = jnp.exp(logits - m)
    s = jnp.sum(e, axis=1, keepdims=True)
    p = e / s
    probs_ref[...] = p
    cls_ref[...] = (p[:, 1:2] > p[:, 0:1]).astype(jnp.int32)


@jax.jit
def _mlp(h, w1t, b1, w2t, b2):
    grid = B // BT
    return pl.pallas_call(
        _mlp_body,
        grid=(grid,),
        in_specs=[
            pl.BlockSpec((BT, D), lambda i: (i, 0)),
            pl.BlockSpec((D, D), lambda i: (0, 0)),
            pl.BlockSpec((1, D), lambda i: (0, 0)),
            pl.BlockSpec((D, NUM_CLASSES), lambda i: (0, 0)),
            pl.BlockSpec((1, NUM_CLASSES), lambda i: (0, 0)),
        ],
        out_specs=[
            pl.BlockSpec((BT, NUM_CLASSES), lambda i: (i, 0)),
            pl.BlockSpec((BT, 1), lambda i: (i, 0)),
        ],
        out_shape=[
            jax.ShapeDtypeStruct((B, NUM_CLASSES), jnp.float32),
            jax.ShapeDtypeStruct((B, 1), jnp.int32),
        ],
    )(h, w1t, b1, w2t, b2)


def kernel(x, table, W1, b1, W2, b2):
    tab2 = _transpose(table.T)
    xp = jnp.where(x < H, x * 2, (x - H) * 2 + 1)
    x2d = xp.reshape(B * L // G, G)
    h = _gather_mean(x2d, tab2.reshape(2 * H, D))
    probs, cls = _mlp(h, W1.T, b1.reshape(1, D), W2.T, b2.reshape(1, NUM_CLASSES))
    return probs, cls.reshape(B)


# speedup vs baseline: 1.3731x; 1.3731x over previous
import jax
import jax.numpy as jnp
from jax import lax
from jax.experimental import pallas as pl
from jax.experimental.pallas import tpu as pltpu
from jax.experimental.pallas import tpu_sc as plsc

B = 16384
L = 200
VOCAB = 1000000
D = 64
NUM_CLASSES = 2

NC = 2
NS = 16
NW = NC * NS

TVC = 512
H = 500224
NT = H // TVC


def _tr_body(a_ref, b_ref, i_ref, out_ref):
    x2 = jnp.concatenate([a_ref[...], b_ref[...]], axis=0)
    out_ref[...] = jax.lax.dot_general(
        x2, i_ref[...], (((0,), (0,)), ((), ())),
        precision=jax.lax.Precision.HIGHEST,
        preferred_element_type=jnp.float32,
    )


@jax.jit
def _transpose(tabt):
    return pl.pallas_call(
        _tr_body,
        grid=(NT,),
        in_specs=[
            pl.BlockSpec((D, TVC), lambda j: (0, j)),
            pl.BlockSpec((D, TVC), lambda j: (0, j + NT)),
            pl.BlockSpec((2 * D, 2 * D), lambda j: (0, 0)),
        ],
        out_specs=pl.BlockSpec((TVC, 2 * D), lambda j: (j, 0)),
        out_shape=jax.ShapeDtypeStruct((H, 2 * D), jnp.float32),
    )(tabt, tabt, jnp.eye(2 * D, dtype=jnp.float32))


RPW = B // NW
CB = 4
G = 100
GPB = L // G
NGC = CB * GPB
NCHUNK = RPW // CB


def _gather_mean_body(x_hbm, tab_hbm, out_hbm, idx_v, rows_v, acc_v, sem0, sem1):
    cid = lax.axis_index("c")
    sid = lax.axis_index("s")
    wid = sid * NC + cid
    base = wid * RPW
    sems = (sem0, sem1)

    def issue(c, p):
        b0 = base + c * CB
        pltpu.sync_copy(x_hbm.at[pl.ds(GPB * b0, NGC)], idx_v.at[p])
        for g in range(NGC):
            pltpu.async_copy(
                tab_hbm.at[idx_v.at[p, g]],
                rows_v.at[p, pl.ds(g * G, G)],
                sems[p],
            )

    def wait_all(p):
        for g in range(NGC):
            pltpu.make_async_copy(
                tab_hbm.at[idx_v.at[p, g]],
                rows_v.at[p, pl.ds(g * G, G)],
                sems[p],
            ).wait()

    def compute(c, p):
        b0 = base + c * CB
        for b in range(CB):
            accs = [jnp.zeros((16,), jnp.float32) for _ in range(4)]

            def red(jj, a, b=b, p=p):
                r = b * L + jj * 4
                out = []
                for q in range(4):
                    r0 = rows_v[p, r, pl.ds(q * 16, 16)]
                    r1 = rows_v[p, r + 1, pl.ds(q * 16, 16)]
                    r2 = rows_v[p, r + 2, pl.ds(q * 16, 16)]
                    r3 = rows_v[p, r + 3, pl.ds(q * 16, 16)]
                    out.append(a[q] + ((r0 + r1) + (r2 + r3)))
                return out

            accs = lax.fori_loop(0, L // 4, red, accs)
            for q in range(4):
                acc_v[b, pl.ds(q * 16, 16)] = accs[q] / jnp.float32(L)
        pltpu.sync_copy(acc_v, out_hbm.at[pl.ds(b0, CB)])

    issue(0, 0)

    def body(i, carry):
        c0 = 2 * i
        issue(c0 + 1, 1)
        wait_all(0)
        compute(c0, 0)

        @pl.when(c0 + 2 < NCHUNK)
        def _():
            issue(c0 + 2, 0)

        wait_all(1)
        compute(c0 + 1, 1)
        return carry

    lax.fori_loop(0, NCHUNK // 2, body, 0)


@jax.jit
def _gather_mean(x2d, table):
    mesh = plsc.VectorSubcoreMesh(core_axis_name="c", subcore_axis_name="s")
    f = pl.kernel(
        _gather_mean_body,
        out_type=jax.ShapeDtypeStruct((B, D), jnp.float32),
        mesh=mesh,
        scratch_types=[
            pltpu.VMEM((2, NGC, G), jnp.int32),
            pltpu.VMEM((2, CB * L, D), jnp.float32),
            pltpu.VMEM((CB, D), jnp.float32),
            pltpu.SemaphoreType.DMA,
            pltpu.SemaphoreType.DMA,
        ],
        compiler_params=pltpu.CompilerParams(use_tc_tiling_on_sc=False),
    )
    return f(x2d, table)


BT = 2048


def _mlp_body(h_ref, w1t_ref, b1_ref, w2t_ref, b2_ref, probs_ref, cls_ref):
    h = h_ref[...]
    z = jnp.tanh(jnp.dot(h, w1t_ref[...]) + b1_ref[...])
    logits = jnp.tanh(jnp.dot(z, w2t_ref[...]) + b2_ref[...])
    m = jnp.max(logits, axis=1, keepdims=True)
    e = jnp.exp(logits - m)
    s = jnp.sum(e, axis=1, keepdims=True)
    p = e / s
    probs_ref[...] = p
    cls_ref[...] = (p[:, 1:2] > p[:, 0:1]).astype(jnp.int32)


@jax.jit
def _mlp(h, w1t, b1, w2t, b2):
    grid = B // BT
    return pl.pallas_call(
        _mlp_body,
        grid=(grid,),
        in_specs=[
            pl.BlockSpec((BT, D), lambda i: (i, 0)),
            pl.BlockSpec((D, D), lambda i: (0, 0)),
            pl.BlockSpec((1, D), lambda i: (0, 0)),
            pl.BlockSpec((D, NUM_CLASSES), lambda i: (0, 0)),
            pl.BlockSpec((1, NUM_CLASSES), lambda i: (0, 0)),
        ],
        out_specs=[
            pl.BlockSpec((BT, NUM_CLASSES), lambda i: (i, 0)),
            pl.BlockSpec((BT, 1), lambda i: (i, 0)),
        ],
        out_shape=[
            jax.ShapeDtypeStruct((B, NUM_CLASSES), jnp.float32),
            jax.ShapeDtypeStruct((B, 1), jnp.int32),
        ],
    )(h, w1t, b1, w2t, b2)


def kernel(x, table, W1, b1, W2, b2):
    tab2 = _transpose(table.T)
    xp = jnp.where(x < H, x * 2, (x - H) * 2 + 1)
    x2d = xp.reshape(B * L // G, G)
    h = _gather_mean(x2d, tab2.reshape(2 * H, D))
    probs, cls = _mlp(h, W1.T, b1.reshape(1, D), W2.T, b2.reshape(1, NUM_CLASSES))
    return probs, cls.reshape(B)


# speedup vs baseline: 2.0598x; 1.5001x over previous
import jax
import jax.numpy as jnp
from jax import lax
from jax.experimental import pallas as pl
from jax.experimental.pallas import tpu as pltpu
from jax.experimental.pallas import tpu_sc as plsc

B = 16384
L = 200
VOCAB = 1000000
D = 64
NUM_CLASSES = 2

NC = 2
NS = 16
NW = NC * NS

TVC = 2048
H = 501760
NT = H // TVC
NBV = VOCAB // TVC


def _tr_body(a_ref, b_ref, i_ref, out_ref):
    x2 = jnp.concatenate([a_ref[...], b_ref[...]], axis=0)
    out_ref[...] = jax.lax.dot_general(
        x2, i_ref[...], (((0,), (0,)), ((), ())),
        precision=jax.lax.Precision.HIGHEST,
        preferred_element_type=jnp.float32,
    )


@jax.jit
def _transpose(tabt):
    return pl.pallas_call(
        _tr_body,
        grid=(NT,),
        in_specs=[
            pl.BlockSpec((D, TVC), lambda j: (0, j)),
            pl.BlockSpec(
                (D, TVC), lambda j: (0, jnp.minimum(j + NT, NBV))
            ),
            pl.BlockSpec((2 * D, 2 * D), lambda j: (0, 0)),
        ],
        out_specs=pl.BlockSpec((TVC, 2 * D), lambda j: (j, 0)),
        out_shape=jax.ShapeDtypeStruct((H, 2 * D), jnp.float32),
    )(tabt, tabt, jnp.eye(2 * D, dtype=jnp.float32))


RPW = B // NW
CB = 4
G = 100
GPB = L // G
NGC = CB * GPB
NCHUNK = RPW // CB


def _gather_mean_body(x_hbm, tab_hbm, out_hbm, idx_v, rows_v, acc_v, sem0, sem1):
    cid = lax.axis_index("c")
    sid = lax.axis_index("s")
    wid = sid * NC + cid
    base = wid * RPW
    sems = (sem0, sem1)

    def issue(c, p):
        b0 = base + c * CB
        pltpu.sync_copy(x_hbm.at[pl.ds(GPB * b0, NGC)], idx_v.at[p])
        for g in range(NGC):
            pltpu.async_copy(
                tab_hbm.at[idx_v.at[p, g]],
                rows_v.at[p, pl.ds(g * G, G)],
                sems[p],
            )

    def wait_all(p):
        for g in range(NGC):
            pltpu.make_async_copy(
                tab_hbm.at[idx_v.at[p, g]],
                rows_v.at[p, pl.ds(g * G, G)],
                sems[p],
            ).wait()

    def compute(c, p):
        b0 = base + c * CB
        for b in range(CB):
            accs = [jnp.zeros((16,), jnp.float32) for _ in range(4)]

            def red(jj, a, b=b, p=p):
                r = b * L + jj * 4
                out = []
                for q in range(4):
                    r0 = rows_v[p, r, pl.ds(q * 16, 16)]
                    r1 = rows_v[p, r + 1, pl.ds(q * 16, 16)]
                    r2 = rows_v[p, r + 2, pl.ds(q * 16, 16)]
                    r3 = rows_v[p, r + 3, pl.ds(q * 16, 16)]
                    out.append(a[q] + ((r0 + r1) + (r2 + r3)))
                return out

            accs = lax.fori_loop(0, L // 4, red, accs)
            for q in range(4):
                acc_v[b, pl.ds(q * 16, 16)] = accs[q] / jnp.float32(L)
        pltpu.sync_copy(acc_v, out_hbm.at[pl.ds(b0, CB)])

    issue(0, 0)

    def body(i, carry):
        c0 = 2 * i
        issue(c0 + 1, 1)
        wait_all(0)
        compute(c0, 0)

        @pl.when(c0 + 2 < NCHUNK)
        def _():
            issue(c0 + 2, 0)

        wait_all(1)
        compute(c0 + 1, 1)
        return carry

    lax.fori_loop(0, NCHUNK // 2, body, 0)


@jax.jit
def _gather_mean(x2d, table):
    mesh = plsc.VectorSubcoreMesh(core_axis_name="c", subcore_axis_name="s")
    f = pl.kernel(
        _gather_mean_body,
        out_type=jax.ShapeDtypeStruct((B, D), jnp.float32),
        mesh=mesh,
        scratch_types=[
            pltpu.VMEM((2, NGC, G), jnp.int32),
            pltpu.VMEM((2, CB * L, D), jnp.float32),
            pltpu.VMEM((CB, D), jnp.float32),
            pltpu.SemaphoreType.DMA,
            pltpu.SemaphoreType.DMA,
        ],
        compiler_params=pltpu.CompilerParams(use_tc_tiling_on_sc=False),
    )
    return f(x2d, table)


BT = 2048


def _mlp_body(h_ref, w1t_ref, b1_ref, w2t_ref, b2_ref, probs_ref, cls_ref):
    h = h_ref[...]
    z = jnp.tanh(jnp.dot(h, w1t_ref[...]) + b1_ref[...])
    logits = jnp.tanh(jnp.dot(z, w2t_ref[...]) + b2_ref[...])
    m = jnp.max(logits, axis=1, keepdims=True)
    e = jnp.exp(logits - m)
    s = jnp.sum(e, axis=1, keepdims=True)
    p = e / s
    probs_ref[...] = p
    cls_ref[...] = (p[:, 1:2] > p[:, 0:1]).astype(jnp.int32)


@jax.jit
def _mlp(h, w1t, b1, w2t, b2):
    grid = B // BT
    return pl.pallas_call(
        _mlp_body,
        grid=(grid,),
        in_specs=[
            pl.BlockSpec((BT, D), lambda i: (i, 0)),
            pl.BlockSpec((D, D), lambda i: (0, 0)),
            pl.BlockSpec((1, D), lambda i: (0, 0)),
            pl.BlockSpec((D, NUM_CLASSES), lambda i: (0, 0)),
            pl.BlockSpec((1, NUM_CLASSES), lambda i: (0, 0)),
        ],
        out_specs=[
            pl.BlockSpec((BT, NUM_CLASSES), lambda i: (i, 0)),
            pl.BlockSpec((BT, 1), lambda i: (i, 0)),
        ],
        out_shape=[
            jax.ShapeDtypeStruct((B, NUM_CLASSES), jnp.float32),
            jax.ShapeDtypeStruct((B, 1), jnp.int32),
        ],
    )(h, w1t, b1, w2t, b2)


def kernel(x, table, W1, b1, W2, b2):
    tab2 = _transpose(table.T)
    xp = jnp.where(x < H, x * 2, (x - H) * 2 + 1)
    x2d = xp.reshape(B * L // G, G)
    h = _gather_mean(x2d, tab2.reshape(2 * H, D))
    probs, cls = _mlp(h, W1.T, b1.reshape(1, D), W2.T, b2.reshape(1, NUM_CLASSES))
    return probs, cls.reshape(B)
